# baseline (speedup 1.0000x reference)
import functools

import jax
import jax.numpy as jnp
from jax.experimental import pallas as pl


_E_BLK = 5000


def _mlp_body(xd_ref, xs_ref, w1_ref, b1_ref, w2_ref, b2_ref, out_ref):
    xd = xd_ref[...]
    xs = xs_ref[...]
    m = jnp.concatenate([xd, xs - xd], axis=1)
    m = jnp.dot(m, w1_ref[...], preferred_element_type=jnp.float32) + b1_ref[...]
    m = jnp.maximum(m, 0.0)
    out_ref[...] = (
        jnp.dot(m, w2_ref[...], preferred_element_type=jnp.float32) + b2_ref[...]
    )


def _edge_mlp(xd, xs, w1, b1, w2, b2):
    e, f = xd.shape
    hdim = w1.shape[1]
    o = w2.shape[1]
    grid = e // _E_BLK
    return pl.pallas_call(
        _mlp_body,
        grid=(grid,),
        in_specs=[
            pl.BlockSpec((_E_BLK, f), lambda i: (i, 0)),
            pl.BlockSpec((_E_BLK, f), lambda i: (i, 0)),
            pl.BlockSpec((2 * f, hdim), lambda i: (0, 0)),
            pl.BlockSpec((1, hdim), lambda i: (0, 0)),
            pl.BlockSpec((hdim, o), lambda i: (0, 0)),
            pl.BlockSpec((1, o), lambda i: (0, 0)),
        ],
        out_specs=pl.BlockSpec((_E_BLK, o), lambda i: (i, 0)),
        out_shape=jax.ShapeDtypeStruct((e, o), jnp.float32),
    )(xd, xs, w1, b1.reshape(1, hdim), w2, b2.reshape(1, o))


def _layer(h, src, dst, w1, b1, w2, b2, relu_out):
    n = h.shape[0]
    xd = jnp.take(h, dst, axis=0)
    xs = jnp.take(h, src, axis=0)
    msg = _edge_mlp(xd, xs, w1, b1, w2, b2)
    out = jax.ops.segment_max(msg, dst, num_segments=n)
    if relu_out:
        return jnp.maximum(out, 0.0)
    return jnp.where(jnp.isfinite(out), out, 0.0)


@jax.jit
def _run(x, edge_index, edge_attr, Wi1, bi1, Wi2, bi2, Wh1, bh1, Wh2, bh2,
         Wo1, bo1, Wo2, bo2):
    src = edge_index[0]
    dst = edge_index[1]
    h = _layer(x, src, dst, Wi1, bi1, Wi2, bi2, relu_out=True)
    for l in range(Wh1.shape[0]):
        h = _layer(h, src, dst, Wh1[l], bh1[l], Wh2[l], bh2[l], relu_out=True)
    out = _layer(h, src, dst, Wo1, bo1, Wo2, bo2, relu_out=False)
    return (out, edge_attr)


def kernel(x, edge_index, edge_attr, Wi1, bi1, Wi2, bi2, Wh1, bh1, Wh2, bh2,
           Wo1, bo1, Wo2, bo2):
    return _run(x, edge_index, edge_attr, Wi1, bi1, Wi2, bi2, Wh1, bh1,
                Wh2, bh2, Wo1, bo1, Wo2, bo2)


# speedup vs baseline: 2.1643x; 2.1643x over previous
import functools

import jax
import jax.numpy as jnp
from jax import lax
from jax.experimental import pallas as pl
from jax.experimental.pallas import tpu as pltpu
from jax.experimental.pallas import tpu_sc as plsc


_E_BLK = 5000

_NC = 2
_NS = 16
_NW = _NC * _NS

_E = 800000
_EW = _E // _NW
_GW = 1000
_GSUB = 125
_GNSUB = _GW // _GSUB
_NWIN = _EW // _GW

_SC_MESH = plsc.VectorSubcoreMesh(core_axis_name="c", subcore_axis_name="s")


def _gather_body(h_hbm, dst3, src3, xd_hbm, xs_hbm, idxd, idxs, bufd, bufs, sem):
    wid = lax.axis_index("s") * _NC + lax.axis_index("c")

    @pl.loop(0, _NWIN)
    def _win(w):
        widx = wid * _NWIN + w
        pltpu.sync_copy(dst3.at[widx], idxd)
        pltpu.sync_copy(src3.at[widx], idxs)
        handles = []
        for j in range(_GNSUB):
            handles.append(
                pltpu.async_copy(
                    h_hbm.at[idxd.at[j]], bufd.at[pl.ds(j * _GSUB, _GSUB)], sem
                )
            )
            handles.append(
                pltpu.async_copy(
                    h_hbm.at[idxs.at[j]], bufs.at[pl.ds(j * _GSUB, _GSUB)], sem
                )
            )
        for h in handles:
            h.wait()
        ebase = widx * _GW
        pltpu.sync_copy(bufd, xd_hbm.at[pl.ds(ebase, _GW)])
        pltpu.sync_copy(bufs, xs_hbm.at[pl.ds(ebase, _GW)])


@functools.partial(functools.lru_cache)
def _make_gather(f):
    return pl.kernel(
        _gather_body,
        out_type=(
            jax.ShapeDtypeStruct((_E, f), jnp.float32),
            jax.ShapeDtypeStruct((_E, f), jnp.float32),
        ),
        mesh=_SC_MESH,
        scratch_types=[
            pltpu.VMEM((_GNSUB, _GSUB), jnp.int32),
            pltpu.VMEM((_GNSUB, _GSUB), jnp.int32),
            pltpu.VMEM((_GW, f), jnp.float32),
            pltpu.VMEM((_GW, f), jnp.float32),
            pltpu.SemaphoreType.DMA,
        ],
        compiler_params=pltpu.CompilerParams(use_tc_tiling_on_sc=False),
    )


def _sc_gather(h, dst3, src3):
    return _make_gather(h.shape[1])(h, dst3, src3)


def _mlp_body(xd_ref, xs_ref, w1_ref, b1_ref, w2_ref, b2_ref, out_ref):
    xd = xd_ref[...]
    xs = xs_ref[...]
    m = jnp.concatenate([xd, xs - xd], axis=1)
    m = jnp.dot(m, w1_ref[...], preferred_element_type=jnp.float32) + b1_ref[...]
    m = jnp.maximum(m, 0.0)
    out_ref[...] = (
        jnp.dot(m, w2_ref[...], preferred_element_type=jnp.float32) + b2_ref[...]
    )


def _edge_mlp(xd, xs, w1, b1, w2, b2):
    e, f = xd.shape
    hdim = w1.shape[1]
    o = w2.shape[1]
    grid = e // _E_BLK
    return pl.pallas_call(
        _mlp_body,
        grid=(grid,),
        in_specs=[
            pl.BlockSpec((_E_BLK, f), lambda i: (i, 0)),
            pl.BlockSpec((_E_BLK, f), lambda i: (i, 0)),
            pl.BlockSpec((2 * f, hdim), lambda i: (0, 0)),
            pl.BlockSpec((1, hdim), lambda i: (0, 0)),
            pl.BlockSpec((hdim, o), lambda i: (0, 0)),
            pl.BlockSpec((1, o), lambda i: (0, 0)),
        ],
        out_specs=pl.BlockSpec((_E_BLK, o), lambda i: (i, 0)),
        out_shape=jax.ShapeDtypeStruct((e, o), jnp.float32),
    )(xd, xs, w1, b1.reshape(1, hdim), w2, b2.reshape(1, o))


def _layer(h, src3, dst3, dst, w1, b1, w2, b2, relu_out):
    n = h.shape[0]
    xd, xs = _sc_gather(h, dst3, src3)
    msg = _edge_mlp(xd, xs, w1, b1, w2, b2)
    out = jax.ops.segment_max(msg, dst, num_segments=n)
    if relu_out:
        return jnp.maximum(out, 0.0)
    return jnp.where(jnp.isfinite(out), out, 0.0)


@jax.jit
def _run(x, edge_index, edge_attr, Wi1, bi1, Wi2, bi2, Wh1, bh1, Wh2, bh2,
         Wo1, bo1, Wo2, bo2):
    src = edge_index[0]
    dst = edge_index[1]
    src3 = src.reshape(_E // _GW, _GNSUB, _GSUB)
    dst3 = dst.reshape(_E // _GW, _GNSUB, _GSUB)
    h = _layer(x, src3, dst3, dst, Wi1, bi1, Wi2, bi2, relu_out=True)
    for l in range(Wh1.shape[0]):
        h = _layer(h, src3, dst3, dst, Wh1[l], bh1[l], Wh2[l], bh2[l],
                   relu_out=True)
    out = _layer(h, src3, dst3, dst, Wo1, bo1, Wo2, bo2, relu_out=False)
    return (out, edge_attr)


def kernel(x, edge_index, edge_attr, Wi1, bi1, Wi2, bi2, Wh1, bh1, Wh2, bh2,
           Wo1, bo1, Wo2, bo2):
    return _run(x, edge_index, edge_attr, Wi1, bi1, Wi2, bi2, Wh1, bh1,
                Wh2, bh2, Wo1, bo1, Wo2, bo2)


# speedup vs baseline: 2.7397x; 1.2658x over previous
import functools

import jax
import jax.numpy as jnp
from jax import lax
from jax.experimental import pallas as pl
from jax.experimental.pallas import tpu as pltpu
from jax.experimental.pallas import tpu_sc as plsc


_NC = 2
_NS = 16
_NW = _NC * _NS

_N = 50000
_E = 800000
_NP = 50176
_RPW = _NP // _NW

_CHUNK = 1024
_EPAD = 32 * 26 * 1024

_BW = 4000
_NBW = _E // _BW

_GW = 1024
_GSUB = 128
_GNSUB = _GW // _GSUB
_NWIN = _EPAD // _NW // _GW

_E_BLK = 4096

_SC_MESH = plsc.VectorSubcoreMesh(core_axis_name="c", subcore_axis_name="s")
_SC_PARAMS = pltpu.CompilerParams(
    use_tc_tiling_on_sc=False, needs_layout_passes=False
)
_NEG_INF = float("-inf")


def _lane():
    return lax.iota(jnp.int32, 16)


def _wid():
    return lax.axis_index("s") * _NC + lax.axis_index("c")


def _scalar(vec):
    return jnp.max(vec)


def _lane_scalar(vec, l):
    return jnp.max(jnp.where(_lane() == l, vec, 0))


def _p1_body(dst2, counts_hbm, dwin, obuf):
    wid = _wid()
    lo = wid * _RPW
    hi = lo + _RPW

    def _win(i, cnt):
        iw = lax.rem(i + wid * 7, _NBW)
        pltpu.sync_copy(dst2.at[iw], dwin)

        def _step(j, c):
            v = dwin[pl.ds(j * 16, 16)]
            m = jnp.logical_and(v >= lo, v < hi)
            return c + jnp.where(m, 1, 0)

        return lax.fori_loop(0, _BW // 16, _step, cnt)

    cnt = lax.fori_loop(0, _NBW, _win, jnp.zeros((16,), jnp.int32))
    total = jnp.sum(cnt)
    obuf[...] = jnp.full((16,), total, jnp.int32)
    pltpu.sync_copy(obuf, counts_hbm.at[wid])


_p1 = pl.kernel(
    _p1_body,
    out_type=jax.ShapeDtypeStruct((_NW, 16), jnp.int32),
    mesh=_SC_MESH,
    scratch_types=[
        pltpu.VMEM((_BW,), jnp.int32),
        pltpu.VMEM((16,), jnp.int32),
    ],
    compiler_params=_SC_PARAMS,
)


def _p2_body(dst2, src2, counts_hbm, dstg, srcg, offs_hbm,
             dwin, swin, dacc, sacc, cbuf, obuf):
    wid = _wid()
    lo = wid * _RPW
    hi = lo + _RPW

    pltpu.sync_copy(counts_hbm, cbuf)
    cv0 = jnp.zeros((16,), jnp.int32)
    cv1 = jnp.zeros((16,), jnp.int32)
    for u in range(16):
        cv0 = jnp.where(_lane() == u, cbuf[u], cv0)
        cv1 = jnp.where(_lane() == u, cbuf[u + 16], cv1)
    r0 = ((cv0 + (_CHUNK - 1)) >> 10) << 10
    r1 = ((cv1 + (_CHUNK - 1)) >> 10) << 10
    s0 = plsc.cumsum(r0)
    s1 = plsc.cumsum(r1)
    t0 = _lane_scalar(s0, 15)
    excl0 = s0 - r0
    excl1 = s1 - r1 + t0
    off_lo = _lane_scalar(excl0, 0)
    my_off = jnp.int32(0)
    for u in range(16):
        my_off = lax.select(wid == u, _lane_scalar(excl0, u), my_off)
        my_off = lax.select(wid == u + 16, _lane_scalar(excl1, u), my_off)
    del off_lo
    my_off = pl.multiple_of(my_off, _CHUNK)
    total_pad = pl.multiple_of(_lane_scalar(s1, 15) + t0, _CHUNK)

    @pl.loop(0, (_CHUNK + 32) // 16)
    def _z(i):
        dacc[pl.ds(i * 16, 16)] = jnp.zeros((16,), jnp.int32)
        sacc[pl.ds(i * 16, 16)] = jnp.zeros((16,), jnp.int32)

    def _win(i, carry):
        p, f = carry
        pltpu.sync_copy(dst2.at[i], dwin)
        pltpu.sync_copy(src2.at[i], swin)

        def _step(j, c):
            p, f = c
            v = dwin[pl.ds(j * 16, 16)]
            s = swin[pl.ds(j * 16, 16)]
            m = jnp.logical_and(v >= lo, v < hi)
            rank = plsc.cumsum(jnp.where(m, 1, 0))
            idx = p + rank - 1
            plsc.store_scatter(dacc, [idx], v, mask=m)
            plsc.store_scatter(sacc, [idx], s, mask=m)
            p = p + jnp.max(rank)
            do = p >= _CHUNK

            @pl.when(do)
            def _flush():
                pltpu.sync_copy(
                    dacc.at[pl.ds(0, _CHUNK)],
                    dstg.at[pl.ds(my_off + f * _CHUNK, _CHUNK)],
                )
                pltpu.sync_copy(
                    sacc.at[pl.ds(0, _CHUNK)],
                    srcg.at[pl.ds(my_off + f * _CHUNK, _CHUNK)],
                )
                dacc[pl.ds(0, 16)] = dacc[pl.ds(_CHUNK, 16)]
                sacc[pl.ds(0, 16)] = sacc[pl.ds(_CHUNK, 16)]

            p = jnp.where(do, p - _CHUNK, p)
            f = jnp.where(do, f + 1, f)
            return p, f

        return lax.fori_loop(0, _BW // 16, _step, (p, f))

    p, f = lax.fori_loop(0, _NBW, _win, (jnp.int32(0), jnp.int32(0)))

    @pl.when(p > 0)
    def _tail():
        pltpu.sync_copy(
            dacc.at[pl.ds(0, _CHUNK)],
            dstg.at[pl.ds(my_off + f * _CHUNK, _CHUNK)],
        )
        pltpu.sync_copy(
            sacc.at[pl.ds(0, _CHUNK)],
            srcg.at[pl.ds(my_off + f * _CHUNK, _CHUNK)],
        )

    obuf[...] = jnp.full((16,), my_off, jnp.int32)
    pltpu.sync_copy(obuf, offs_hbm.at[wid])

    @pl.when(wid == _NW - 1)
    def _fill_tail():
        @pl.loop(0, _CHUNK // 16)
        def _zz(i):
            v = (i * 16 + _lane()) & 1023
            dacc[pl.ds(i * 16, 16)] = v
            sacc[pl.ds(i * 16, 16)] = v

        def _fill(g, _):
            pltpu.sync_copy(
                dacc.at[pl.ds(0, _CHUNK)],
                dstg.at[pl.ds(total_pad + g * _CHUNK, _CHUNK)],
            )
            pltpu.sync_copy(
                sacc.at[pl.ds(0, _CHUNK)],
                srcg.at[pl.ds(total_pad + g * _CHUNK, _CHUNK)],
            )
            return 0

        lax.fori_loop(0, (_EPAD - total_pad) >> 10, _fill, 0)


_p2 = pl.kernel(
    _p2_body,
    out_type=(
        jax.ShapeDtypeStruct((_EPAD,), jnp.int32),
        jax.ShapeDtypeStruct((_EPAD,), jnp.int32),
        jax.ShapeDtypeStruct((_NW, 16), jnp.int32),
    ),
    mesh=_SC_MESH,
    scratch_types=[
        pltpu.VMEM((_BW,), jnp.int32),
        pltpu.VMEM((_BW,), jnp.int32),
        pltpu.VMEM((_CHUNK + 32,), jnp.int32),
        pltpu.VMEM((_CHUNK + 32,), jnp.int32),
        pltpu.VMEM((_NW, 16), jnp.int32),
        pltpu.VMEM((16,), jnp.int32),
    ],
    compiler_params=_SC_PARAMS,
)


def _gather_body(h_hbm, dst3, src3, xd_hbm, xs_hbm, idxd, idxs, bufd, bufs, sem):
    wid = _wid()

    @pl.loop(0, _NWIN)
    def _win(w):
        widx = wid * _NWIN + w
        pltpu.sync_copy(dst3.at[widx], idxd)
        pltpu.sync_copy(src3.at[widx], idxs)
        handles = []
        for j in range(_GNSUB):
            handles.append(
                pltpu.async_copy(
                    h_hbm.at[idxd.at[j]], bufd.at[pl.ds(j * _GSUB, _GSUB)], sem
                )
            )
            handles.append(
                pltpu.async_copy(
                    h_hbm.at[idxs.at[j]], bufs.at[pl.ds(j * _GSUB, _GSUB)], sem
                )
            )
        for h in handles:
            h.wait()
        ebase = widx * _GW
        pltpu.sync_copy(bufd, xd_hbm.at[pl.ds(ebase, _GW)])
        pltpu.sync_copy(bufs, xs_hbm.at[pl.ds(ebase, _GW)])


@functools.lru_cache
def _make_gather(f):
    return pl.kernel(
        _gather_body,
        out_type=(
            jax.ShapeDtypeStruct((_EPAD, f), jnp.float32),
            jax.ShapeDtypeStruct((_EPAD, f), jnp.float32),
        ),
        mesh=_SC_MESH,
        scratch_types=[
            pltpu.VMEM((_GNSUB, _GSUB), jnp.int32),
            pltpu.VMEM((_GNSUB, _GSUB), jnp.int32),
            pltpu.VMEM((_GW, f), jnp.float32),
            pltpu.VMEM((_GW, f), jnp.float32),
            pltpu.SemaphoreType.DMA,
        ],
        compiler_params=_SC_PARAMS,
    )


def _scatter_body(hdim, relu_out,
                  msg_hbm, dstg_hbm, counts_hbm, offs_hbm, acc_hbm,
                  mwin, dwin, cbuf, obuf, oloc):
    wid = _wid()
    lo = wid * _RPW
    pltpu.sync_copy(counts_hbm.at[wid], cbuf)
    pltpu.sync_copy(offs_hbm.at[wid], obuf)
    cnt = _scalar(cbuf[...])
    off = pl.multiple_of(_scalar(obuf[...]), _CHUNK)

    ninf = jnp.full((16,), _NEG_INF, jnp.float32)

    @pl.loop(0, _RPW)
    def _init(r):
        for c in range(hdim // 16):
            oloc[r, pl.ds(c * 16, 16)] = ninf

    nwin = (cnt + _CHUNK - 1) >> 10

    def _win(k, _):
        base = off + k * _CHUNK
        pltpu.sync_copy(msg_hbm.at[pl.ds(base, _CHUNK)], mwin)
        pltpu.sync_copy(dstg_hbm.at[pl.ds(base, _CHUNK)], dwin)
        rem = cnt - k * _CHUNK
        kk = jnp.minimum(rem, _CHUNK)
        nfull = kk >> 4
        tail = kk & 15

        def _vreg(jv, _):
            dvec = dwin[pl.ds(jv * 16, 16)] - lo
            for l in range(16):
                d = _lane_scalar(dvec, l)
                row = jv * 16 + l
                for c in range(hdim // 16):
                    cur = oloc[d, pl.ds(c * 16, 16)]
                    mv = mwin[row, pl.ds(c * 16, 16)]
                    oloc[d, pl.ds(c * 16, 16)] = jnp.maximum(cur, mv)
            return 0

        lax.fori_loop(0, nfull, _vreg, 0)

        @pl.when(tail > 0)
        def _tail():
            dvec = dwin[pl.ds(nfull * 16, 16)] - lo
            for l in range(16):
                @pl.when(l < tail)
                def _one():
                    d = _lane_scalar(dvec, l)
                    row = nfull * 16 + l
                    for c in range(hdim // 16):
                        cur = oloc[d, pl.ds(c * 16, 16)]
                        mv = mwin[row, pl.ds(c * 16, 16)]
                        oloc[d, pl.ds(c * 16, 16)] = jnp.maximum(cur, mv)

        return 0

    lax.fori_loop(0, nwin, _win, 0)

    @pl.loop(0, _RPW)
    def _fin(r):
        for c in range(hdim // 16):
            v = oloc[r, pl.ds(c * 16, 16)]
            if relu_out:
                v = jnp.maximum(v, 0.0)
            else:
                v = jnp.where(jnp.abs(v) < jnp.float32(float("inf")), v, 0.0)
            oloc[r, pl.ds(c * 16, 16)] = v

    pltpu.sync_copy(oloc, acc_hbm.at[pl.ds(lo, _RPW)])


@functools.lru_cache
def _make_scatter(hdim, relu_out):
    return pl.kernel(
        functools.partial(_scatter_body, hdim, relu_out),
        out_type=jax.ShapeDtypeStruct((_NP, hdim), jnp.float32),
        mesh=_SC_MESH,
        scratch_types=[
            pltpu.VMEM((_CHUNK, hdim), jnp.float32),
            pltpu.VMEM((_CHUNK,), jnp.int32),
            pltpu.VMEM((16,), jnp.int32),
            pltpu.VMEM((16,), jnp.int32),
            pltpu.VMEM((_RPW, hdim), jnp.float32),
        ],
        compiler_params=_SC_PARAMS,
    )


def _mlp_body(xd_ref, xs_ref, w1_ref, b1_ref, w2_ref, b2_ref, out_ref):
    xd = xd_ref[...]
    xs = xs_ref[...]
    m = jnp.concatenate([xd, xs - xd], axis=1)
    m = jnp.dot(m, w1_ref[...], preferred_element_type=jnp.float32) + b1_ref[...]
    m = jnp.maximum(m, 0.0)
    out_ref[...] = (
        jnp.dot(m, w2_ref[...], preferred_element_type=jnp.float32) + b2_ref[...]
    )


def _edge_mlp(xd, xs, w1, b1, w2, b2):
    e, f = xd.shape
    hdim = w1.shape[1]
    o = w2.shape[1]
    grid = e // _E_BLK
    return pl.pallas_call(
        _mlp_body,
        grid=(grid,),
        in_specs=[
            pl.BlockSpec((_E_BLK, f), lambda i: (i, 0)),
            pl.BlockSpec((_E_BLK, f), lambda i: (i, 0)),
            pl.BlockSpec((2 * f, hdim), lambda i: (0, 0)),
            pl.BlockSpec((1, hdim), lambda i: (0, 0)),
            pl.BlockSpec((hdim, o), lambda i: (0, 0)),
            pl.BlockSpec((1, o), lambda i: (0, 0)),
        ],
        out_specs=pl.BlockSpec((_E_BLK, o), lambda i: (i, 0)),
        out_shape=jax.ShapeDtypeStruct((e, o), jnp.float32),
    )(xd, xs, w1, b1.reshape(1, hdim), w2, b2.reshape(1, o))


def _layer(h, dstg3, srcg3, dstg, counts, offs, w1, b1, w2, b2, relu_out, odim):
    xd, xs = _make_gather(h.shape[1])(h, dstg3, srcg3)
    msg = _edge_mlp(xd, xs, w1, b1, w2, b2)
    return _make_scatter(odim, relu_out)(msg, dstg, counts, offs)


@jax.jit
def _run(x, edge_index, edge_attr, Wi1, bi1, Wi2, bi2, Wh1, bh1, Wh2, bh2,
         Wo1, bo1, Wo2, bo2):
    src = edge_index[0]
    dst = edge_index[1]
    dst2 = dst.reshape(_NBW, _BW)
    src2 = src.reshape(_NBW, _BW)
    counts = _p1(dst2)
    dstg, srcg, offs = _p2(dst2, src2, counts)
    dstg3 = dstg.reshape(_EPAD // _GW, _GNSUB, _GSUB)
    srcg3 = srcg.reshape(_EPAD // _GW, _GNSUB, _GSUB)

    Wo2p = jnp.concatenate([Wo2, jnp.zeros((Wo2.shape[0], 15), jnp.float32)], 1)
    bo2p = jnp.concatenate([bo2, jnp.zeros((15,), jnp.float32)])

    h = _layer(x, dstg3, srcg3, dstg, counts, offs, Wi1, bi1, Wi2, bi2,
               relu_out=True, odim=32)
    for l in range(Wh1.shape[0]):
        h = _layer(h, dstg3, srcg3, dstg, counts, offs,
                   Wh1[l], bh1[l], Wh2[l], bh2[l], relu_out=True, odim=32)
    acc = _layer(h, dstg3, srcg3, dstg, counts, offs, Wo1, bo1, Wo2p, bo2p,
                 relu_out=False, odim=16)
    out = acc[:_N, 0:1]
    return (out, edge_attr)


def kernel(x, edge_index, edge_attr, Wi1, bi1, Wi2, bi2, Wh1, bh1, Wh2, bh2,
           Wo1, bo1, Wo2, bo2):
    return _run(x, edge_index, edge_attr, Wi1, bi1, Wi2, bi2, Wh1, bh1,
                Wh2, bh2, Wo1, bo1, Wo2, bo2)


# speedup vs baseline: 3.1135x; 1.1364x over previous
import functools

import jax
import jax.numpy as jnp
from jax import lax
from jax.experimental import pallas as pl
from jax.experimental.pallas import tpu as pltpu
from jax.experimental.pallas import tpu_sc as plsc


_NC = 2
_NS = 16
_NW = _NC * _NS

_N = 50000
_E = 800000
_NP = 50176
_RPW = _NP // _NW

_CHUNK = 1024
_EPAD = 32 * 26 * 1024

_BW = 8000
_NBW = _E // _BW
_PGRP = 10
_ACC = _CHUNK + 192

_GW = 512
_GSUB = 128
_GNSUB = _GW // _GSUB
_NWIN = _EPAD // _NW // _GW

_E_BLK = 4096

_SC_MESH = plsc.VectorSubcoreMesh(core_axis_name="c", subcore_axis_name="s")
_SC_PARAMS = pltpu.CompilerParams(
    use_tc_tiling_on_sc=False, needs_layout_passes=False
)
_NEG_INF = float("-inf")


def _lane():
    return lax.iota(jnp.int32, 16)


def _wid():
    return lax.axis_index("s") * _NC + lax.axis_index("c")


def _scalar(vec):
    return jnp.max(vec)


def _lane_scalar(vec, l):
    return jnp.max(jnp.where(_lane() == l, vec, 0))


def _p1_body(dst2, counts_hbm, dwin, obuf):
    wid = _wid()
    lo = wid * _RPW
    hi = lo + _RPW

    def _win(i, cnt):
        iw = lax.rem(i + wid * 7, _NBW)
        pltpu.sync_copy(dst2.at[iw], dwin)

        def _step(j, c):
            v = dwin[pl.ds(j * 16, 16)]
            m = jnp.logical_and(v >= lo, v < hi)
            return c + jnp.where(m, 1, 0)

        return lax.fori_loop(0, _BW // 16, _step, cnt)

    cnt = lax.fori_loop(0, _NBW, _win, jnp.zeros((16,), jnp.int32))
    total = jnp.sum(cnt)
    obuf[...] = jnp.full((16,), total, jnp.int32)
    pltpu.sync_copy(obuf, counts_hbm.at[wid])


_p1 = pl.kernel(
    _p1_body,
    out_type=jax.ShapeDtypeStruct((_NW, 16), jnp.int32),
    mesh=_SC_MESH,
    scratch_types=[
        pltpu.VMEM((_BW,), jnp.int32),
        pltpu.VMEM((16,), jnp.int32),
    ],
    compiler_params=_SC_PARAMS,
)


def _p2_body(dst2, src2, counts_hbm, dstg, srcg, offs_hbm,
             dwin0, swin0, dwin1, swin1, dacc, sacc, cbuf, obuf,
             semw0, semw1):
    wid = _wid()
    lo = wid * _RPW
    hi = lo + _RPW

    pltpu.sync_copy(counts_hbm, cbuf)
    cv0 = jnp.zeros((16,), jnp.int32)
    cv1 = jnp.zeros((16,), jnp.int32)
    for u in range(16):
        cv0 = jnp.where(_lane() == u, cbuf[u], cv0)
        cv1 = jnp.where(_lane() == u, cbuf[u + 16], cv1)
    r0 = ((cv0 + (_CHUNK - 1)) >> 10) << 10
    r1 = ((cv1 + (_CHUNK - 1)) >> 10) << 10
    s0 = plsc.cumsum(r0)
    s1 = plsc.cumsum(r1)
    t0 = _lane_scalar(s0, 15)
    excl0 = s0 - r0
    excl1 = s1 - r1 + t0
    off_lo = _lane_scalar(excl0, 0)
    my_off = jnp.int32(0)
    for u in range(16):
        my_off = lax.select(wid == u, _lane_scalar(excl0, u), my_off)
        my_off = lax.select(wid == u + 16, _lane_scalar(excl1, u), my_off)
    del off_lo
    my_off = pl.multiple_of(my_off, _CHUNK)
    total_pad = pl.multiple_of(_lane_scalar(s1, 15) + t0, _CHUNK)

    @pl.loop(0, _ACC // 16)
    def _z(i):
        dacc[pl.ds(i * 16, 16)] = jnp.zeros((16,), jnp.int32)
        sacc[pl.ds(i * 16, 16)] = jnp.zeros((16,), jnp.int32)

    dwin = (dwin0, dwin1)
    swin = (swin0, swin1)
    semw = (semw0, semw1)

    def _issue_win(i, b):
        pltpu.async_copy(dst2.at[i], dwin[b], semw[b])
        pltpu.async_copy(src2.at[i], swin[b], semw[b])

    def _wait_win(b):
        pltpu.make_async_copy(dst2.at[0], dwin[b], semw[b]).wait()
        pltpu.make_async_copy(src2.at[0], swin[b], semw[b]).wait()

    def _scan_win(b, carry):
        def _grp(g, c):
            p, f = c
            for t in range(_PGRP):
                j = g * _PGRP + t
                v = dwin[b][pl.ds(j * 16, 16)]
                s = swin[b][pl.ds(j * 16, 16)]
                m = jnp.logical_and(v >= lo, v < hi)
                rank = plsc.cumsum(jnp.where(m, 1, 0))
                idx = p + rank - 1
                plsc.store_scatter(dacc, [idx], v, mask=m)
                plsc.store_scatter(sacc, [idx], s, mask=m)
                p = p + plsc.all_reduce_population_count(m)
            ps = jnp.max(p)
            do = ps >= _CHUNK

            @pl.when(do)
            def _flush():
                pltpu.sync_copy(
                    dacc.at[pl.ds(0, _CHUNK)],
                    dstg.at[pl.ds(my_off + f * _CHUNK, _CHUNK)],
                )
                pltpu.sync_copy(
                    sacc.at[pl.ds(0, _CHUNK)],
                    srcg.at[pl.ds(my_off + f * _CHUNK, _CHUNK)],
                )
                for t in range((_ACC - _CHUNK) // 16):
                    dacc[pl.ds(t * 16, 16)] = dacc[pl.ds(_CHUNK + t * 16, 16)]
                    sacc[pl.ds(t * 16, 16)] = sacc[pl.ds(_CHUNK + t * 16, 16)]

            p = jnp.where(do, p - _CHUNK, p)
            f = jnp.where(do, f + 1, f)
            return p, f

        return lax.fori_loop(0, _BW // 16 // _PGRP, _grp, carry)

    p0 = jnp.zeros((16,), jnp.int32)
    _issue_win(0, 0)
    _issue_win(1, 1)
    _wait_win(0)
    carry = _scan_win(0, (p0, jnp.int32(0)))
    _issue_win(2, 0)
    _wait_win(1)
    carry = _scan_win(1, carry)
    _issue_win(3, 1)

    def _wloop(k, carry):
        ia = 2 * k
        _wait_win(0)
        carry = _scan_win(0, carry)
        na = jnp.minimum(ia + 2, _NBW - 1)
        _issue_win(na, 0)
        _wait_win(1)
        carry = _scan_win(1, carry)
        nb = jnp.minimum(ia + 3, _NBW - 1)
        _issue_win(nb, 1)
        return carry

    pv, f = lax.fori_loop(1, _NBW // 2, _wloop, carry)
    _wait_win(0)
    _wait_win(1)
    p = jnp.max(pv)

    @pl.when(p > 0)
    def _tail():
        pltpu.sync_copy(
            dacc.at[pl.ds(0, _CHUNK)],
            dstg.at[pl.ds(my_off + f * _CHUNK, _CHUNK)],
        )
        pltpu.sync_copy(
            sacc.at[pl.ds(0, _CHUNK)],
            srcg.at[pl.ds(my_off + f * _CHUNK, _CHUNK)],
        )

    obuf[...] = jnp.full((16,), my_off, jnp.int32)
    pltpu.sync_copy(obuf, offs_hbm.at[wid])

    @pl.when(wid == _NW - 1)
    def _fill_tail():
        @pl.loop(0, _CHUNK // 16)
        def _zz(i):
            v = (i * 16 + _lane()) & 1023
            dacc[pl.ds(i * 16, 16)] = v
            sacc[pl.ds(i * 16, 16)] = v

        def _fill(g, _):
            pltpu.sync_copy(
                dacc.at[pl.ds(0, _CHUNK)],
                dstg.at[pl.ds(total_pad + g * _CHUNK, _CHUNK)],
            )
            pltpu.sync_copy(
                sacc.at[pl.ds(0, _CHUNK)],
                srcg.at[pl.ds(total_pad + g * _CHUNK, _CHUNK)],
            )
            return 0

        lax.fori_loop(0, (_EPAD - total_pad) >> 10, _fill, 0)


_p2 = pl.kernel(
    _p2_body,
    out_type=(
        jax.ShapeDtypeStruct((_EPAD,), jnp.int32),
        jax.ShapeDtypeStruct((_EPAD,), jnp.int32),
        jax.ShapeDtypeStruct((_NW, 16), jnp.int32),
    ),
    mesh=_SC_MESH,
    scratch_types=[
        pltpu.VMEM((_BW,), jnp.int32),
        pltpu.VMEM((_BW,), jnp.int32),
        pltpu.VMEM((_BW,), jnp.int32),
        pltpu.VMEM((_BW,), jnp.int32),
        pltpu.VMEM((_ACC,), jnp.int32),
        pltpu.VMEM((_ACC,), jnp.int32),
        pltpu.VMEM((_NW, 16), jnp.int32),
        pltpu.VMEM((16,), jnp.int32),
        pltpu.SemaphoreType.DMA,
        pltpu.SemaphoreType.DMA,
    ],
    compiler_params=_SC_PARAMS,
)


def _gather_body(h_hbm, dst3, src3, xd_hbm, xs_hbm,
                 idd0, ids0, idd1, ids1, bd0, bs0, bd1, bs1,
                 semi0, semi1, semg, semo0, semo1):
    wid = _wid()
    base = wid * _NWIN
    idd = (idd0, idd1)
    ids = (ids0, ids1)
    bd = (bd0, bd1)
    bs = (bs0, bs1)
    semi = (semi0, semi1)
    semo = (semo0, semo1)

    def _issue_idx(widx, b):
        pltpu.async_copy(dst3.at[widx], idd[b], semi[b])
        pltpu.async_copy(src3.at[widx], ids[b], semi[b])

    def _wait_idx(b):
        pltpu.make_async_copy(dst3.at[0], idd[b], semi[b]).wait()
        pltpu.make_async_copy(src3.at[0], ids[b], semi[b]).wait()

    def _gathers(b):
        hs = []
        for j in range(_GNSUB):
            hs.append(pltpu.async_copy(
                h_hbm.at[idd[b].at[j]], bd[b].at[pl.ds(j * _GSUB, _GSUB)], semg))
            hs.append(pltpu.async_copy(
                h_hbm.at[ids[b].at[j]], bs[b].at[pl.ds(j * _GSUB, _GSUB)], semg))
        for h in hs:
            h.wait()

    def _issue_wb(widx, b):
        ebase = widx * _GW
        pltpu.async_copy(bd[b], xd_hbm.at[pl.ds(ebase, _GW)], semo[b])
        pltpu.async_copy(bs[b], xs_hbm.at[pl.ds(ebase, _GW)], semo[b])

    def _wait_wb(b):
        pltpu.make_async_copy(bd[b], xd_hbm.at[pl.ds(0, _GW)], semo[b]).wait()
        pltpu.make_async_copy(bs[b], xs_hbm.at[pl.ds(0, _GW)], semo[b]).wait()

    _issue_idx(base, 0)
    _issue_idx(base + 1, 1)
    _wait_idx(0)
    _gathers(0)
    _issue_wb(base, 0)
    _issue_idx(base + 2, 0)
    _wait_idx(1)
    _gathers(1)
    _issue_wb(base + 1, 1)
    _issue_idx(base + 3, 1)

    @pl.loop(1, _NWIN // 2)
    def _k(k):
        wa = base + 2 * k
        _wait_idx(0)
        _wait_wb(0)
        _gathers(0)
        _issue_wb(wa, 0)
        na = jnp.minimum(wa + 2, base + _NWIN - 1)
        _issue_idx(na, 0)
        _wait_idx(1)
        _wait_wb(1)
        _gathers(1)
        _issue_wb(wa + 1, 1)
        nb = jnp.minimum(wa + 3, base + _NWIN - 1)
        _issue_idx(nb, 1)

    _wait_idx(0)
    _wait_idx(1)
    _wait_wb(0)
    _wait_wb(1)


@functools.lru_cache
def _make_gather(f):
    return pl.kernel(
        _gather_body,
        out_type=(
            jax.ShapeDtypeStruct((_EPAD, f), jnp.float32),
            jax.ShapeDtypeStruct((_EPAD, f), jnp.float32),
        ),
        mesh=_SC_MESH,
        scratch_types=[
            pltpu.VMEM((_GNSUB, _GSUB), jnp.int32),
            pltpu.VMEM((_GNSUB, _GSUB), jnp.int32),
            pltpu.VMEM((_GNSUB, _GSUB), jnp.int32),
            pltpu.VMEM((_GNSUB, _GSUB), jnp.int32),
            pltpu.VMEM((_GW, f), jnp.float32),
            pltpu.VMEM((_GW, f), jnp.float32),
            pltpu.VMEM((_GW, f), jnp.float32),
            pltpu.VMEM((_GW, f), jnp.float32),
            pltpu.SemaphoreType.DMA,
            pltpu.SemaphoreType.DMA,
            pltpu.SemaphoreType.DMA,
            pltpu.SemaphoreType.DMA,
            pltpu.SemaphoreType.DMA,
        ],
        compiler_params=_SC_PARAMS,
    )


def _scatter_body(hdim, relu_out,
                  msg_hbm, dstg_hbm, counts_hbm, offs_hbm, acc_hbm,
                  mwin, dwin, cbuf, obuf, oloc):
    wid = _wid()
    lo = wid * _RPW
    pltpu.sync_copy(counts_hbm.at[wid], cbuf)
    pltpu.sync_copy(offs_hbm.at[wid], obuf)
    cnt = _scalar(cbuf[...])
    off = pl.multiple_of(_scalar(obuf[...]), _CHUNK)

    ninf = jnp.full((16,), _NEG_INF, jnp.float32)

    @pl.loop(0, _RPW)
    def _init(r):
        for c in range(hdim // 16):
            oloc[r, pl.ds(c * 16, 16)] = ninf

    nwin = (cnt + _CHUNK - 1) >> 10

    def _win(k, _):
        base = off + k * _CHUNK
        pltpu.sync_copy(msg_hbm.at[pl.ds(base, _CHUNK)], mwin)
        pltpu.sync_copy(dstg_hbm.at[pl.ds(base, _CHUNK)], dwin)
        rem = cnt - k * _CHUNK
        kk = jnp.minimum(rem, _CHUNK)
        nfull = kk >> 4
        tail = kk & 15

        def _vreg(jv, _):
            dvec = dwin[pl.ds(jv * 16, 16)] - lo
            for l in range(16):
                d = _lane_scalar(dvec, l)
                row = jv * 16 + l
                for c in range(hdim // 16):
                    cur = oloc[d, pl.ds(c * 16, 16)]
                    mv = mwin[row, pl.ds(c * 16, 16)]
                    oloc[d, pl.ds(c * 16, 16)] = jnp.maximum(cur, mv)
            return 0

        lax.fori_loop(0, nfull, _vreg, 0)

        @pl.when(tail > 0)
        def _tail():
            dvec = dwin[pl.ds(nfull * 16, 16)] - lo
            for l in range(16):
                @pl.when(l < tail)
                def _one():
                    d = _lane_scalar(dvec, l)
                    row = nfull * 16 + l
                    for c in range(hdim // 16):
                        cur = oloc[d, pl.ds(c * 16, 16)]
                        mv = mwin[row, pl.ds(c * 16, 16)]
                        oloc[d, pl.ds(c * 16, 16)] = jnp.maximum(cur, mv)

        return 0

    lax.fori_loop(0, nwin, _win, 0)

    @pl.loop(0, _RPW)
    def _fin(r):
        for c in range(hdim // 16):
            v = oloc[r, pl.ds(c * 16, 16)]
            if relu_out:
                v = jnp.maximum(v, 0.0)
            else:
                v = jnp.where(jnp.abs(v) < jnp.float32(float("inf")), v, 0.0)
            oloc[r, pl.ds(c * 16, 16)] = v

    pltpu.sync_copy(oloc, acc_hbm.at[pl.ds(lo, _RPW)])


@functools.lru_cache
def _make_scatter(hdim, relu_out):
    return pl.kernel(
        functools.partial(_scatter_body, hdim, relu_out),
        out_type=jax.ShapeDtypeStruct((_NP, hdim), jnp.float32),
        mesh=_SC_MESH,
        scratch_types=[
            pltpu.VMEM((_CHUNK, hdim), jnp.float32),
            pltpu.VMEM((_CHUNK,), jnp.int32),
            pltpu.VMEM((16,), jnp.int32),
            pltpu.VMEM((16,), jnp.int32),
            pltpu.VMEM((_RPW, hdim), jnp.float32),
        ],
        compiler_params=_SC_PARAMS,
    )


def _mlp_body(xd_ref, xs_ref, w1_ref, b1_ref, w2_ref, b2_ref, out_ref):
    xd = xd_ref[...]
    xs = xs_ref[...]
    m = jnp.concatenate([xd, xs - xd], axis=1)
    m = jnp.dot(m, w1_ref[...], preferred_element_type=jnp.float32) + b1_ref[...]
    m = jnp.maximum(m, 0.0)
    out_ref[...] = (
        jnp.dot(m, w2_ref[...], preferred_element_type=jnp.float32) + b2_ref[...]
    )


def _edge_mlp(xd, xs, w1, b1, w2, b2):
    e, f = xd.shape
    hdim = w1.shape[1]
    o = w2.shape[1]
    grid = e // _E_BLK
    return pl.pallas_call(
        _mlp_body,
        grid=(grid,),
        in_specs=[
            pl.BlockSpec((_E_BLK, f), lambda i: (i, 0)),
            pl.BlockSpec((_E_BLK, f), lambda i: (i, 0)),
            pl.BlockSpec((2 * f, hdim), lambda i: (0, 0)),
            pl.BlockSpec((1, hdim), lambda i: (0, 0)),
            pl.BlockSpec((hdim, o), lambda i: (0, 0)),
            pl.BlockSpec((1, o), lambda i: (0, 0)),
        ],
        out_specs=pl.BlockSpec((_E_BLK, o), lambda i: (i, 0)),
        out_shape=jax.ShapeDtypeStruct((e, o), jnp.float32),
    )(xd, xs, w1, b1.reshape(1, hdim), w2, b2.reshape(1, o))


def _layer(h, dstg3, srcg3, dstg, counts, offs, w1, b1, w2, b2, relu_out, odim):
    xd, xs = _make_gather(h.shape[1])(h, dstg3, srcg3)
    msg = _edge_mlp(xd, xs, w1, b1, w2, b2)
    return _make_scatter(odim, relu_out)(msg, dstg, counts, offs)


@jax.jit
def _run(x, edge_index, edge_attr, Wi1, bi1, Wi2, bi2, Wh1, bh1, Wh2, bh2,
         Wo1, bo1, Wo2, bo2):
    src = edge_index[0]
    dst = edge_index[1]
    dst2 = dst.reshape(_NBW, _BW)
    src2 = src.reshape(_NBW, _BW)
    counts = _p1(dst2)
    dstg, srcg, offs = _p2(dst2, src2, counts)
    dstg3 = dstg.reshape(_EPAD // _GW, _GNSUB, _GSUB)
    srcg3 = srcg.reshape(_EPAD // _GW, _GNSUB, _GSUB)

    Wo2p = jnp.concatenate([Wo2, jnp.zeros((Wo2.shape[0], 15), jnp.float32)], 1)
    bo2p = jnp.concatenate([bo2, jnp.zeros((15,), jnp.float32)])

    h = _layer(x, dstg3, srcg3, dstg, counts, offs, Wi1, bi1, Wi2, bi2,
               relu_out=True, odim=32)
    for l in range(Wh1.shape[0]):
        h = _layer(h, dstg3, srcg3, dstg, counts, offs,
                   Wh1[l], bh1[l], Wh2[l], bh2[l], relu_out=True, odim=32)
    acc = _layer(h, dstg3, srcg3, dstg, counts, offs, Wo1, bo1, Wo2p, bo2p,
                 relu_out=False, odim=16)
    out = acc[:_N, 0:1]
    return (out, edge_attr)


def kernel(x, edge_index, edge_attr, Wi1, bi1, Wi2, bi2, Wh1, bh1, Wh2, bh2,
           Wo1, bo1, Wo2, bo2):
    return _run(x, edge_index, edge_attr, Wi1, bi1, Wi2, bi2, Wh1, bh1,
                Wh2, bh2, Wo1, bo1, Wo2, bo2)
